# baseline (speedup 1.0000x reference)
import jax
import jax.numpy as jnp
from jax.experimental import pallas as pl
from jax.experimental.pallas import tpu as pltpu

_SMOOTHING = 0.1
_CONF = 1.0 - _SMOOTHING
_NCLS = 1000
_EPS = _SMOOTHING / (_NCLS - 1)

_ROWS = 16384
_BLK = 1024
_GRID = _ROWS // _BLK


def _loss_kernel(pred_ref, tgt_ref, out_ref):
    i = pl.program_id(0)
    x = pred_ref[...]
    t = tgt_ref[...]

    rowmax = jnp.max(x, axis=1, keepdims=True)
    sumexp = jnp.sum(jnp.exp(x - rowmax), axis=1)
    lse = rowmax[:, 0] + jnp.log(sumexp)

    cols = jax.lax.broadcasted_iota(jnp.int32, x.shape, 1)
    w = jnp.where(cols == t, _CONF, _EPS)
    wsum = jnp.sum(x * w, axis=1)

    part = (jnp.sum(lse - wsum) * (1.0 / _ROWS)).reshape(1, 1)

    @pl.when(i == 0)
    def _init():
        out_ref[...] = jnp.zeros_like(out_ref)

    out_ref[...] += part


def kernel(pred, target):
    tgt2 = target.astype(jnp.int32).reshape(_ROWS, 1)
    out = pl.pallas_call(
        _loss_kernel,
        grid=(_GRID,),
        in_specs=[
            pl.BlockSpec((_BLK, _NCLS), lambda i: (i, 0)),
            pl.BlockSpec((_BLK, 1), lambda i: (i, 0)),
        ],
        out_specs=pl.BlockSpec((1, 1), lambda i: (0, 0)),
        out_shape=jax.ShapeDtypeStruct((1, 1), jnp.float32),
        compiler_params=pltpu.CompilerParams(
            dimension_semantics=("arbitrary",),
        ),
    )(pred, tgt2)
    return out[0, 0]


# speedup vs baseline: 1.0123x; 1.0123x over previous
import jax
import jax.numpy as jnp
from jax.experimental import pallas as pl
from jax.experimental.pallas import tpu as pltpu

_SMOOTHING = 0.1
_CONF = 1.0 - _SMOOTHING
_NCLS = 1000
_EPS = _SMOOTHING / (_NCLS - 1)

_ROWS = 16384
_BLK = 1024
_HALF = _ROWS // 2
_GRID = _HALF // _BLK


def _half_loss(x, t):
    rowmax = jnp.max(x, axis=1, keepdims=True)
    sumexp = jnp.sum(jnp.exp(x - rowmax), axis=1)
    lse = rowmax[:, 0] + jnp.log(sumexp)
    sump = jnp.sum(x, axis=1)

    cols = jax.lax.broadcasted_iota(jnp.int32, x.shape, 1)
    ptar = jnp.sum(jnp.where(cols == t, x, 0.0), axis=1)

    return jnp.sum(lse - _EPS * sump - (_CONF - _EPS) * ptar)


def _loss_kernel(p1_ref, p2_ref, t1_ref, t2_ref, out_ref):
    i = pl.program_id(0)
    part = ((_half_loss(p1_ref[...], t1_ref[...])
             + _half_loss(p2_ref[...], t2_ref[...]))
            * (1.0 / _ROWS)).reshape(1, 1)

    @pl.when(i == 0)
    def _init():
        out_ref[...] = jnp.zeros_like(out_ref)

    out_ref[...] += part


def kernel(pred, target):
    tgt2 = target.astype(jnp.int32).reshape(_ROWS, 1)
    out = pl.pallas_call(
        _loss_kernel,
        grid=(_GRID,),
        in_specs=[
            pl.BlockSpec((_BLK, _NCLS), lambda i: (i, 0)),
            pl.BlockSpec((_BLK, _NCLS), lambda i: (i + _GRID, 0)),
            pl.BlockSpec((_BLK, 1), lambda i: (i, 0)),
            pl.BlockSpec((_BLK, 1), lambda i: (i + _GRID, 0)),
        ],
        out_specs=pl.BlockSpec((1, 1), lambda i: (0, 0)),
        out_shape=jax.ShapeDtypeStruct((1, 1), jnp.float32),
        compiler_params=pltpu.CompilerParams(
            dimension_semantics=("arbitrary",),
        ),
    )(pred, pred, tgt2, tgt2)
    return out[0, 0]


# speedup vs baseline: 1.0579x; 1.0451x over previous
import jax
import jax.numpy as jnp
from jax.experimental import pallas as pl
from jax.experimental.pallas import tpu as pltpu

_SMOOTHING = 0.1
_CONF = 1.0 - _SMOOTHING
_NCLS = 1000
_EPS = _SMOOTHING / (_NCLS - 1)

_ROWS = 16384
_BLK = 1024
_GRID = _ROWS // _BLK


def _loss_kernel(pred_ref, tgt_ref, out_ref):
    i = pl.program_id(0)
    x = pred_ref[...]
    t = tgt_ref[0, 0, :]

    rowmax = jnp.max(x, axis=1, keepdims=True)
    sumexp = jnp.sum(jnp.exp(x - rowmax), axis=1)
    lse = rowmax[:, 0] + jnp.log(sumexp)
    sump = jnp.sum(x, axis=1)

    cols = jax.lax.broadcasted_iota(jnp.int32, x.shape, 1)
    onehot = cols == t[:, None]
    ptar = jnp.sum(jnp.where(onehot, x, 0.0), axis=1)

    part = (jnp.sum(lse - _EPS * sump - (_CONF - _EPS) * ptar)
            * (1.0 / _ROWS)).reshape(1, 1)

    @pl.when(i == 0)
    def _init():
        out_ref[...] = jnp.zeros_like(out_ref)

    out_ref[...] += part


def kernel(pred, target):
    tgt3 = target.astype(jnp.int32).reshape(_GRID, 1, _BLK)
    out = pl.pallas_call(
        _loss_kernel,
        grid=(_GRID,),
        in_specs=[
            pl.BlockSpec((_BLK, _NCLS), lambda i: (i, 0)),
            pl.BlockSpec((1, 1, _BLK), lambda i: (i, 0, 0)),
        ],
        out_specs=pl.BlockSpec((1, 1), lambda i: (0, 0)),
        out_shape=jax.ShapeDtypeStruct((1, 1), jnp.float32),
        compiler_params=pltpu.CompilerParams(
            dimension_semantics=("arbitrary",),
        ),
    )(pred, tgt3)
    return out[0, 0]


# speedup vs baseline: 1.0608x; 1.0028x over previous
import jax
import jax.numpy as jnp
from jax.experimental import pallas as pl
from jax.experimental.pallas import tpu as pltpu

_SMOOTHING = 0.1
_CONF = 1.0 - _SMOOTHING
_NCLS = 1000
_EPS = _SMOOTHING / (_NCLS - 1)

_ROWS = 16384
_BLK = 1024
_GRID = _ROWS // _BLK


def _loss_kernel(pred_ref, tgt_ref, out_ref):
    i = pl.program_id(0)
    x = pred_ref[...]
    t = tgt_ref[0, 0, :]

    rowmax = jnp.max(x, axis=1, keepdims=True)
    sumexp = jnp.sum(jnp.exp(x - rowmax), axis=1)
    lse = rowmax[:, 0] + jnp.log(sumexp)
    tsum = jnp.sum(x)

    cols = jax.lax.broadcasted_iota(jnp.int32, x.shape, 1)
    onehot = cols == t[:, None]
    ptar = jnp.sum(jnp.where(onehot, x, 0.0), axis=1)

    part = ((jnp.sum(lse - (_CONF - _EPS) * ptar) - _EPS * tsum)
            * (1.0 / _ROWS)).reshape(1, 1)

    @pl.when(i == 0)
    def _init():
        out_ref[...] = jnp.zeros_like(out_ref)

    out_ref[...] += part


def kernel(pred, target):
    tgt3 = target.astype(jnp.int32).reshape(_GRID, 1, _BLK)
    out = pl.pallas_call(
        _loss_kernel,
        grid=(_GRID,),
        in_specs=[
            pl.BlockSpec((_BLK, _NCLS), lambda i: (i, 0)),
            pl.BlockSpec((1, 1, _BLK), lambda i: (i, 0, 0)),
        ],
        out_specs=pl.BlockSpec((1, 1), lambda i: (0, 0)),
        out_shape=jax.ShapeDtypeStruct((1, 1), jnp.float32),
        compiler_params=pltpu.CompilerParams(
            dimension_semantics=("arbitrary",),
        ),
    )(pred, tgt3)
    return out[0, 0]
